# Initial kernel scaffold; baseline (speedup 1.0000x reference)
import jax
import jax.numpy as jnp
from jax.experimental import pallas as pl


def kernel(ids, W, b):
    raise NotImplementedError("write your pallas kernel here")


# speedup vs baseline: 32.7168x; 32.7168x over previous
import functools

import jax
import jax.numpy as jnp
from jax import lax
from jax.experimental import pallas as pl
from jax.experimental.pallas import tpu as pltpu
from jax.experimental.pallas import tpu_sc as plsc

_B, _L, _V, _C = 4096, 200, 1000, 9
_CP = 16
_NW = 32
_ROWS_PER_W = _B // _NW
_GROUPS = _ROWS_PER_W // 16
_IDS_PER_W = _ROWS_PER_W * _L


def _sc_accumulate(ids_flat, wt):
    mesh = plsc.VectorSubcoreMesh(core_axis_name="c", subcore_axis_name="s")

    @functools.partial(
        pl.kernel,
        mesh=mesh,
        compiler_params=pltpu.CompilerParams(needs_layout_passes=False),
        out_type=jax.ShapeDtypeStruct((_NW, _CP, _ROWS_PER_W), jnp.float32),
        scratch_types=[
            pltpu.VMEM((_IDS_PER_W,), jnp.int32),
            pltpu.VMEM((_CP, _V), jnp.float32),
            pltpu.VMEM((_CP, _ROWS_PER_W), jnp.float32),
        ],
    )
    def sc_kernel(ids_hbm, wt_hbm, out_hbm, ids_v, wt_v, out_v):
        num_c = lax.axis_size("c")
        wid = lax.axis_index("s") * num_c + lax.axis_index("c")
        pltpu.sync_copy(ids_hbm.at[pl.ds(wid * _IDS_PER_W, _IDS_PER_W)], ids_v)
        pltpu.sync_copy(wt_hbm, wt_v)
        lane = lax.iota(jnp.int32, 16)
        cvecs = [jnp.full((16,), c, jnp.int32) for c in range(_C)]
        for g in range(_GROUPS):
            base = lane * _L + g * (16 * _L)

            def jbody(j, accs, base=base):
                idsj = plsc.load_gather(ids_v, [base + j])
                return tuple(
                    accs[c] + plsc.load_gather(wt_v, [cvecs[c], idsj])
                    for c in range(_C)
                )

            accs = lax.fori_loop(
                0, _L, jbody,
                tuple(jnp.zeros((16,), jnp.float32) for _ in range(_C)),
            )
            for c in range(_C):
                out_v[c, pl.ds(g * 16, 16)] = accs[c]
            for c in range(_C, _CP):
                out_v[c, pl.ds(g * 16, 16)] = jnp.zeros((16,), jnp.float32)
        pltpu.sync_copy(out_v, out_hbm.at[wid])

    return sc_kernel(ids_flat, wt)


def _tc_log_softmax(s, b_tile):
    def body(s_ref, b_ref, o_ref):
        logits = s_ref[0] + b_ref[...]
        cls = lax.broadcasted_iota(jnp.int32, (_CP, _ROWS_PER_W), 0)
        valid = cls < _C
        m = jnp.max(jnp.where(valid, logits, -1e30), axis=0, keepdims=True)
        e = jnp.where(valid, jnp.exp(logits - m), 0.0)
        lse = jnp.log(jnp.sum(e, axis=0, keepdims=True))
        o_ref[0] = logits - m - lse

    return pl.pallas_call(
        body,
        grid=(_NW,),
        in_specs=[
            pl.BlockSpec((1, _CP, _ROWS_PER_W), lambda i: (i, 0, 0)),
            pl.BlockSpec((_CP, _ROWS_PER_W), lambda i: (0, 0)),
        ],
        out_specs=pl.BlockSpec((1, _CP, _ROWS_PER_W), lambda i: (i, 0, 0)),
        out_shape=jax.ShapeDtypeStruct((_NW, _CP, _ROWS_PER_W), jnp.float32),
    )(s, b_tile)


def kernel(ids, W, b):
    ids_flat = ids.reshape(-1)
    wt = jnp.zeros((_CP, _V), jnp.float32).at[:_C].set(W)
    b_pad = jnp.zeros((_CP,), jnp.float32).at[:_C].set(b)
    b_tile = jnp.broadcast_to(b_pad[:, None], (_CP, _ROWS_PER_W))
    s = _sc_accumulate(ids_flat, wt)
    out3 = _tc_log_softmax(s, b_tile)
    return out3.transpose(0, 2, 1).reshape(_B, _CP)[:, :_C]


# speedup vs baseline: 36.1042x; 1.1035x over previous
import functools

import jax
import jax.numpy as jnp
from jax import lax
from jax.experimental import pallas as pl
from jax.experimental.pallas import tpu as pltpu
from jax.experimental.pallas import tpu_sc as plsc

_B, _L, _V, _C = 4096, 200, 1000, 9
_VP = 1024
_CP = 16
_NW = 32
_ROWS_PER_W = _B // _NW
_IDS_PER_W = _ROWS_PER_W * _L


def _sc_accumulate(ids_flat, table):
    mesh = plsc.VectorSubcoreMesh(core_axis_name="c", subcore_axis_name="s")

    @functools.partial(
        pl.kernel,
        mesh=mesh,
        compiler_params=pltpu.CompilerParams(needs_layout_passes=False),
        out_type=jax.ShapeDtypeStruct((_B * _CP,), jnp.float32),
        scratch_types=[
            pltpu.VMEM((_IDS_PER_W,), jnp.int32),
            pltpu.VMEM((_VP * _CP,), jnp.float32),
            pltpu.VMEM((_ROWS_PER_W * _CP,), jnp.float32),
        ],
    )
    def sc_kernel(ids_hbm, tab_hbm, out_hbm, ids_v, tab_v, out_v):
        num_c = lax.axis_size("c")
        wid = lax.axis_index("s") * num_c + lax.axis_index("c")
        pltpu.sync_copy(ids_hbm.at[pl.ds(wid * _IDS_PER_W, _IDS_PER_W)], ids_v)
        pltpu.sync_copy(tab_hbm, tab_v)

        def row_body(r, _):
            base = r * _L

            def j_body(jj, accs):
                vec = ids_v[pl.ds(base + jj * 16, 16)]
                for u in range(16):
                    accs = (accs[1:]) + (accs[0] + tab_v[pl.ds(vec[u] * _CP, _CP)],)
                return accs

            accs = lax.fori_loop(
                0, _L // 16, j_body,
                tuple(jnp.zeros((_CP,), jnp.float32) for _ in range(4)),
            )
            vec = ids_v[pl.ds(base + _L - 16, 16)]
            for u in range(8, 16):
                accs = (accs[1:]) + (accs[0] + tab_v[pl.ds(vec[u] * _CP, _CP)],)
            out_v[pl.ds(r * _CP, _CP)] = (accs[0] + accs[1]) + (accs[2] + accs[3])
            return 0

        lax.fori_loop(0, _ROWS_PER_W, row_body, 0)
        pltpu.sync_copy(
            out_v, out_hbm.at[pl.ds(wid * _ROWS_PER_W * _CP, _ROWS_PER_W * _CP)])

    return sc_kernel(ids_flat, table)


def _tc_log_softmax(s, b_row):
    def body(s_ref, b_ref, o_ref):
        logits = s_ref[...] + b_ref[...]
        cls = lax.broadcasted_iota(jnp.int32, (_B, _CP), 1)
        valid = cls < _C
        m = jnp.max(jnp.where(valid, logits, -1e30), axis=1, keepdims=True)
        e = jnp.where(valid, jnp.exp(logits - m), 0.0)
        lse = jnp.log(jnp.sum(e, axis=1, keepdims=True))
        o_ref[...] = (logits - m - lse)[:, :_C]

    return pl.pallas_call(
        body,
        out_shape=jax.ShapeDtypeStruct((_B, _C), jnp.float32),
    )(s, b_row)


def kernel(ids, W, b):
    ids_flat = ids.reshape(-1)
    table = jnp.zeros((_VP, _CP), jnp.float32).at[:_V, :_C].set(W.T).reshape(-1)
    b_row = jnp.pad(b, (0, _CP - _C))[None, :]
    s = _sc_accumulate(ids_flat, table).reshape(_B, _CP)
    return _tc_log_softmax(s, b_row)


# speedup vs baseline: 40.6626x; 1.1263x over previous
import functools

import jax
import jax.numpy as jnp
from jax import lax
from jax.experimental import pallas as pl
from jax.experimental.pallas import tpu as pltpu
from jax.experimental.pallas import tpu_sc as plsc

_B, _L, _V, _C = 4096, 200, 1000, 9
_VP = 1024
_CP = 16
_NW = 32
_ROWS_PER_W = _B // _NW
_IDS_PER_W = _ROWS_PER_W * _L


def _sc_accumulate(ids_flat, table):
    mesh = plsc.VectorSubcoreMesh(core_axis_name="c", subcore_axis_name="s")

    @functools.partial(
        pl.kernel,
        mesh=mesh,
        compiler_params=pltpu.CompilerParams(needs_layout_passes=False),
        out_type=jax.ShapeDtypeStruct((_B * 128,), jnp.float32),
        scratch_types=[
            pltpu.VMEM((_ROWS_PER_W, _L), jnp.int32),
            pltpu.VMEM((_VP * _CP,), jnp.float32),
            pltpu.VMEM((_ROWS_PER_W * 128,), jnp.float32),
        ],
    )
    def sc_kernel(ids_hbm, tab_hbm, out_hbm, ids_v, tab_v, out_v):
        num_c = lax.axis_size("c")
        wid = lax.axis_index("s") * num_c + lax.axis_index("c")
        pltpu.sync_copy(ids_hbm.at[pl.ds(wid * _ROWS_PER_W, _ROWS_PER_W)], ids_v)
        pltpu.sync_copy(tab_hbm, tab_v)

        def row_body(r, _):

            def j_body(jj, accs):
                vec = ids_v[r, pl.ds(jj * 16, 16)]
                for u in range(16):
                    accs = (accs[1:]) + (accs[0] + tab_v[pl.ds(vec[u] * _CP, _CP)],)
                return accs

            accs = lax.fori_loop(
                0, _L // 16, j_body,
                tuple(jnp.zeros((_CP,), jnp.float32) for _ in range(4)),
            )
            vec = ids_v[r, pl.ds(_L - 16, 16)]
            for u in range(8, 16):
                accs = (accs[1:]) + (accs[0] + tab_v[pl.ds(vec[u] * _CP, _CP)],)
            out_v[pl.ds(r * 128, _CP)] = (accs[0] + accs[1]) + (accs[2] + accs[3])
            return 0

        lax.fori_loop(0, _ROWS_PER_W, row_body, 0)
        pltpu.sync_copy(
            out_v, out_hbm.at[pl.ds(wid * _ROWS_PER_W * 128, _ROWS_PER_W * 128)])

    return sc_kernel(ids_flat, table)


def _tc_log_softmax(s, b_row):
    def body(s_ref, b_ref, o_ref):
        logits = s_ref[:, :_CP] + b_ref[...]
        cls = lax.broadcasted_iota(jnp.int32, (_B, _CP), 1)
        valid = cls < _C
        m = jnp.max(jnp.where(valid, logits, -1e30), axis=1, keepdims=True)
        e = jnp.where(valid, jnp.exp(logits - m), 0.0)
        lse = jnp.log(jnp.sum(e, axis=1, keepdims=True))
        o_ref[...] = (logits - m - lse)[:, :_C]

    return pl.pallas_call(
        body,
        out_shape=jax.ShapeDtypeStruct((_B, _C), jnp.float32),
    )(s, b_row)


def kernel(ids, W, b):
    table = jnp.zeros((_VP, _CP), jnp.float32).at[:_V, :_C].set(W.T).reshape(-1)
    b_row = jnp.pad(b, (0, _CP - _C))[None, :]
    s = _sc_accumulate(ids, table).reshape(_B, 128)
    return _tc_log_softmax(s, b_row)
